# Initial kernel scaffold; baseline (speedup 1.0000x reference)
import jax
import jax.numpy as jnp
from jax.experimental import pallas as pl


def kernel(x, W0, W1, W2):
    raise NotImplementedError("write your pallas kernel here")


# speedup vs baseline: 21.2983x; 21.2983x over previous
import functools

import jax
import jax.numpy as jnp
import numpy as np
from jax import lax
from jax.experimental import pallas as pl
from jax.experimental.pallas import tpu as pltpu
from jax.experimental.pallas import tpu_sc as plsc

N_CH = 100
RESOLUTIONS = (16, 64, 256)
DIM = 16
BATCH = 16384

NUM_CORES = 2
NUM_SUBCORES = 16
NW = NUM_CORES * NUM_SUBCORES
B_PER_W = BATCH // NW
CHUNK_B = 32
ROWS = CHUNK_B * N_CH
N_CHUNKS = B_PER_W // CHUNK_B
LANES = 16
N_VEC = ROWS // LANES


def _make_kernel():
    mesh = plsc.VectorSubcoreMesh(core_axis_name="c", subcore_axis_name="s")

    @functools.partial(
        pl.kernel,
        out_type=jax.ShapeDtypeStruct((BATCH * N_CH, 3 * DIM), jnp.float32),
        mesh=mesh,
        compiler_params=pltpu.CompilerParams(use_tc_tiling_on_sc=False),
        scratch_types=[
            pltpu.VMEM((ROWS,), jnp.float32),
            pltpu.VMEM((ROWS,), jnp.int32),
            pltpu.VMEM((ROWS,), jnp.int32),
            pltpu.VMEM((ROWS, DIM), jnp.float32),
            pltpu.SemaphoreType.DMA,
        ],
    )
    def mre_kernel(x_hbm, w0_hbm, w1_hbm, w2_hbm, out_hbm,
                   x_v, ch_v, idx_v, rows_v, sem):
        wid = lax.axis_index("s") * NUM_CORES + lax.axis_index("c")
        base_row = wid * (B_PER_W * N_CH)
        tables = (w0_hbm, w1_hbm, w2_hbm)

        def ch_body(i, _):
            v = lax.iota(jnp.int32, LANES) + i * LANES
            ch_v[pl.ds(i * LANES, LANES)] = lax.rem(v, N_CH)
            return 0

        lax.fori_loop(0, N_VEC, ch_body, 0)

        def chunk_body(chunk, _):
            off = base_row + chunk * ROWS
            pltpu.sync_copy(x_hbm.at[pl.ds(off, ROWS)], x_v)
            for ri, res in enumerate(RESOLUTIONS):
                scale = np.float32(res - 1)
                step = np.float32(1.0 / (res - 1))

                def idx_body(i, _, res=res, scale=scale, step=step):
                    s = pl.ds(i * LANES, LANES)
                    xv = x_v[s]
                    t = xv * scale
                    c0 = t.astype(jnp.int32)
                    base = c0 - 1
                    acc = jnp.maximum(base, 0)
                    for j in range(4):
                        k = base + j
                        bk = k.astype(jnp.float32) * step
                        valid = (k >= 0) & (k <= res - 1)
                        acc = acc + jnp.where(valid & (bk < xv),
                                              jnp.int32(1), jnp.int32(0))
                    idx_v[s] = acc + ch_v[s] * (res + 1)
                    return 0

                lax.fori_loop(0, N_VEC, idx_body, 0)
                pltpu.async_copy(tables[ri].at[idx_v], rows_v, sem).wait()
                pltpu.sync_copy(
                    rows_v, out_hbm.at[pl.ds(off, ROWS), pl.ds(ri * DIM, DIM)])
            return 0

        lax.fori_loop(0, N_CHUNKS, chunk_body, 0)

    return mre_kernel


_MRE = _make_kernel()


def kernel(x, W0, W1, W2):
    out = _MRE(x.reshape(-1), W0, W1, W2)
    return out.reshape(BATCH, N_CH, 3 * DIM)
